# baseline (speedup 1.0000x reference)
import functools

import jax
import jax.numpy as jnp
from jax import lax
from jax.experimental import pallas as pl
from jax.experimental.pallas import tpu as pltpu
from jax.experimental.pallas import tpu_sc as plsc

_DIM = 32
_BATCH = 16384
_N_FIELDS = 26
_B = _BATCH * _N_FIELDS
_NC, _NS = 2, 16
_NW = _NC * _NS
_BPW = _B // _NW
_C = 1024
_NCHUNK = _BPW // _C
_BLK = 128
_MPC = _C // _BLK
_KT = _DIM // 8
_NBB = _BATCH // _BLK
_BPWB = _BPW // _BLK

_mesh = plsc.VectorSubcoreMesh(
    core_axis_name="c", subcore_axis_name="s", num_cores=_NC, num_subcores=_NS
)

_NEMB = 1000000
_SLABC = 512
_NSLAB = _NEMB // _SLABC
_SLABR = _NSLAB - (_NSLAB // _NW) * _NW


@functools.partial(
    pl.kernel,
    out_type=jax.ShapeDtypeStruct((_NEMB * _DIM,), jnp.float32),
    mesh=_mesh,
    scratch_types=[
        pltpu.VMEM((_DIM, _SLABC), jnp.float32),
        pltpu.VMEM((_DIM, _SLABC), jnp.float32),
        pltpu.VMEM((_DIM, _SLABC), jnp.float32),
        pltpu.VMEM((_SLABC * _DIM,), jnp.float32),
        pltpu.VMEM((_SLABC * _DIM,), jnp.float32),
        pltpu.VMEM((_SLABC * _DIM,), jnp.float32),
        pltpu.VMEM((2048,), jnp.float32),
        pltpu.SemaphoreType.DMA,
        pltpu.SemaphoreType.DMA,
        pltpu.SemaphoreType.DMA,
        pltpu.SemaphoreType.DMA,
        pltpu.SemaphoreType.DMA,
        pltpu.SemaphoreType.DMA,
    ],
    compiler_params=pltpu.CompilerParams(
        use_tc_tiling_on_sc=True, needs_layout_passes=False),
)
def _relayout_kernel(wt_hbm, tail_hbm, out_hbm, tb0, tb1, tb2, ab0, ab1,
                     ab2, tlb, sg0, sg1, sg2, ss0, ss1, ss2):
    wid = lax.axis_index("s") * _NC + lax.axis_index("c")
    lane = lax.broadcasted_iota(jnp.int32, (16,), 0)
    tbs = (tb0, tb1, tb2)
    abs_ = (ab0, ab1, ab2)
    gsems = (sg0, sg1, sg2)
    ssems = (ss0, ss1, ss2)
    _ND = 3

    def transpose_slab(tb, ab):
        @plsc.parallel_loop(0, _SLABC, unroll=4)
        def _(p):
            colp = (p + lane) & (_SLABC - 1)
            colp32 = colp << 5
            for h in range(_DIM // 16):
                rowv = h * 16 + lane
                v = plsc.load_gather(tb, [rowv, colp])
                plsc.store_scatter(ab, [colp32 + rowv], v)

    nper = _NSLAB // _NW

    def load_slab(i, parity):
        return pltpu.async_copy(
            wt_hbm.at[:, pl.ds((wid * nper + i) * _SLABC, _SLABC)],
            tbs[parity], gsems[parity])

    def store_slab(i, parity):
        return pltpu.async_copy(
            abs_[parity],
            out_hbm.at[pl.ds((wid * nper + i) * (_SLABC * _DIM),
                             _SLABC * _DIM)],
            ssems[parity])

    gathers = [None] * nper
    stores = [None] * nper
    gathers[0] = load_slab(0, 0)
    gathers[1] = load_slab(1, 1)
    for i in range(nper):
        nxt = i + 2
        if nxt < nper:
            if nxt >= _ND:
                stores[nxt - _ND].wait()
            gathers[nxt] = load_slab(nxt, nxt % _ND)
        gathers[i].wait()
        transpose_slab(tbs[i % _ND], abs_[i % _ND])
        stores[i] = store_slab(i, i % _ND)
    stores[nper - 3].wait()
    stores[nper - 2].wait()
    stores[nper - 1].wait()

    @pl.when(wid == 0)
    def _():
        g = pltpu.async_copy(
            wt_hbm.at[:, pl.ds((_NSLAB - 1) * _SLABC, _SLABC)], tbs[0],
            sg0)
        g.wait()
        transpose_slab(tbs[0], abs_[0])
        pltpu.sync_copy(
            abs_[0],
            out_hbm.at[pl.ds((_NSLAB - 1) * (_SLABC * _DIM),
                             _SLABC * _DIM)])

    @pl.when(wid == 1)
    def _():
        tail = _NEMB - _NSLAB * _SLABC
        pltpu.sync_copy(tail_hbm, tlb)
        pltpu.sync_copy(
            tlb, out_hbm.at[pl.ds(_NSLAB * _SLABC * _DIM, tail * _DIM)])


@functools.partial(
    pl.kernel,
    out_type=jax.ShapeDtypeStruct((_B * _DIM,), jnp.float32),
    mesh=_mesh,
    scratch_types=[
        pltpu.VMEM((_BPW,), jnp.int32),
        pltpu.VMEM((_C, _DIM), jnp.float32),
        pltpu.VMEM((_C, _DIM), jnp.float32),
        pltpu.VMEM((_C * _DIM,), jnp.float32),
        pltpu.SemaphoreType.DMA,
        pltpu.SemaphoreType.DMA,
        pltpu.SemaphoreType.DMA,
    ],
    compiler_params=pltpu.CompilerParams(
        use_tc_tiling_on_sc=False, needs_layout_passes=False),
)
def _gather_kernel(idx_hbm, table_hbm, out_hbm, idx_v, buf0, buf1, abuf,
                   sg0, sg1, ss):
    wid = lax.axis_index("s") * _NC + lax.axis_index("c")
    base = wid * _BPW
    pltpu.sync_copy(idx_hbm.at[pl.ds(base, _BPW)], idx_v)
    bufs = (buf0, buf1)
    gsems = (sg0, sg1)
    lane = lax.broadcasted_iota(jnp.int32, (16,), 0)
    colvs = [(j + lane) & (_DIM - 1) for j in range(_DIM)]

    def transpose_chunk(gbuf):
        @plsc.parallel_loop(0, _C // 8, unroll=1)
        def _(u):
            t = u >> 1
            j0 = (u & 1) * (_DIM // 2)
            rows = t * 16 + lane
            abase = t * 16 + (t >> 3) * (_BLK * _DIM - _BLK)
            for jj in range(_DIM // 2):
                colv = (j0 + jj + lane) & (_DIM - 1)
                v = plsc.load_gather(gbuf, [rows, colv])
                plsc.store_scatter(abuf, [abase + (colv << 7) + lane], v)

    def store_chunk(g):
        beta0 = wid * _BPWB + g * _MPC
        copies = []
        for m in range(_MPC):
            beta = beta0 + m
            f = beta // _NBB
            c = beta - f * _NBB
            for k in range(_KT):
                off = ((f * _KT + k) * _NBB + c) * (8 * _BLK)
                copies.append(pltpu.async_copy(
                    abuf.at[pl.ds(m * (_BLK * _DIM) + k * (8 * _BLK),
                                  8 * _BLK)],
                    out_hbm.at[pl.ds(off, 8 * _BLK)], ss))
        return copies

    gathers = [None] * _NCHUNK
    stores = [None] * _NCHUNK
    gathers[0] = pltpu.async_copy(
        table_hbm.at[idx_v.at[pl.ds(0, _C)]], bufs[0], sg0)
    for g in range(_NCHUNK):
        nxt = g + 1
        if nxt < _NCHUNK:
            gathers[nxt] = pltpu.async_copy(
                table_hbm.at[idx_v.at[pl.ds(nxt * _C, _C)]],
                bufs[nxt % 2], gsems[nxt % 2])
        gathers[g].wait()
        if g >= 1:
            for cp in stores[g - 1]:
                cp.wait()
        transpose_chunk(bufs[g % 2])
        stores[g] = store_chunk(g)
    for cp in stores[_NCHUNK - 1]:
        cp.wait()


def kernel(x, weights):
    idx = x.T.reshape(_B)
    tail = weights[_NSLAB * _SLABC:].reshape(-1)
    tlin = _relayout_kernel(weights.T, tail).reshape(_NEMB, _DIM)
    flat = _gather_kernel(idx, tlin)
    out5 = flat.reshape(_N_FIELDS, _KT, _NBB, 8, _BLK)
    return out5.transpose(2, 4, 0, 1, 3).reshape(_BATCH, _N_FIELDS, _DIM)


# speedup vs baseline: 1.0143x; 1.0143x over previous
import functools

import jax
import jax.numpy as jnp
from jax import lax
from jax.experimental import pallas as pl
from jax.experimental.pallas import tpu as pltpu
from jax.experimental.pallas import tpu_sc as plsc

_DIM = 32
_BATCH = 16384
_N_FIELDS = 26
_B = _BATCH * _N_FIELDS
_NC, _NS = 2, 16
_NW = _NC * _NS
_BPW = _B // _NW
_C = 1024
_NCHUNK = _BPW // _C
_BLK = 128
_MPC = _C // _BLK
_KT = _DIM // 8
_NBB = _BATCH // _BLK
_BPWB = _BPW // _BLK

_mesh = plsc.VectorSubcoreMesh(
    core_axis_name="c", subcore_axis_name="s", num_cores=_NC, num_subcores=_NS
)

_NEMB = 1000000
_SLABC = 512
_NSLAB = _NEMB // _SLABC


@functools.partial(
    pl.kernel,
    out_type=jax.ShapeDtypeStruct((_NEMB * _DIM,), jnp.float32),
    mesh=_mesh,
    scratch_types=[
        pltpu.VMEM((_DIM, _SLABC), jnp.float32),
        pltpu.VMEM((_DIM, _SLABC), jnp.float32),
        pltpu.VMEM((_DIM, _SLABC), jnp.float32),
        pltpu.VMEM((_SLABC * _DIM,), jnp.float32),
        pltpu.VMEM((_SLABC * _DIM,), jnp.float32),
        pltpu.VMEM((_SLABC * _DIM,), jnp.float32),
        pltpu.VMEM((2048,), jnp.float32),
        pltpu.SemaphoreType.DMA,
        pltpu.SemaphoreType.DMA,
        pltpu.SemaphoreType.DMA,
        pltpu.SemaphoreType.DMA,
        pltpu.SemaphoreType.DMA,
        pltpu.SemaphoreType.DMA,
    ],
    compiler_params=pltpu.CompilerParams(
        use_tc_tiling_on_sc=True, needs_layout_passes=False),
)
def _relayout_kernel(wt_hbm, tail_hbm, out_hbm, tb0, tb1, tb2, ab0, ab1,
                     ab2, tlb, sg0, sg1, sg2, ss0, ss1, ss2):
    wid = lax.axis_index("s") * _NC + lax.axis_index("c")
    lane = lax.broadcasted_iota(jnp.int32, (16,), 0)
    tbs = (tb0, tb1, tb2)
    abs_ = (ab0, ab1, ab2)
    gsems = (sg0, sg1, sg2)
    ssems = (ss0, ss1, ss2)
    _ND = 3

    def transpose_slab(tb, ab):
        @plsc.parallel_loop(0, _SLABC, unroll=4)
        def _(p):
            colp = (p + lane) & (_SLABC - 1)
            colp32 = colp << 5
            for h in range(_DIM // 16):
                rowv = h * 16 + lane
                v = plsc.load_gather(tb, [rowv, colp])
                plsc.store_scatter(ab, [colp32 + rowv], v)

    nper = _NSLAB // _NW

    def load_slab(i, parity):
        return pltpu.async_copy(
            wt_hbm.at[:, pl.ds((wid + i * _NW) * _SLABC, _SLABC)],
            tbs[parity], gsems[parity])

    def store_slab(i, parity):
        return pltpu.async_copy(
            abs_[parity],
            out_hbm.at[pl.ds((wid + i * _NW) * (_SLABC * _DIM),
                             _SLABC * _DIM)],
            ssems[parity])

    gathers = [None] * nper
    stores = [None] * nper
    gathers[0] = load_slab(0, 0)
    gathers[1] = load_slab(1, 1)
    for i in range(nper):
        nxt = i + 2
        if nxt < nper:
            if nxt >= _ND:
                stores[nxt - _ND].wait()
            gathers[nxt] = load_slab(nxt, nxt % _ND)
        gathers[i].wait()
        transpose_slab(tbs[i % _ND], abs_[i % _ND])
        stores[i] = store_slab(i, i % _ND)
    stores[nper - 3].wait()
    stores[nper - 2].wait()
    stores[nper - 1].wait()

    @pl.when(wid == 0)
    def _():
        g = pltpu.async_copy(
            wt_hbm.at[:, pl.ds((_NSLAB - 1) * _SLABC, _SLABC)], tbs[0],
            sg0)
        g.wait()
        transpose_slab(tbs[0], abs_[0])
        pltpu.sync_copy(
            abs_[0],
            out_hbm.at[pl.ds((_NSLAB - 1) * (_SLABC * _DIM),
                             _SLABC * _DIM)])

    @pl.when(wid == 1)
    def _():
        tail = _NEMB - _NSLAB * _SLABC
        pltpu.sync_copy(tail_hbm, tlb)
        pltpu.sync_copy(
            tlb, out_hbm.at[pl.ds(_NSLAB * _SLABC * _DIM, tail * _DIM)])


@functools.partial(
    pl.kernel,
    out_type=jax.ShapeDtypeStruct((_B * _DIM,), jnp.float32),
    mesh=_mesh,
    scratch_types=[
        pltpu.VMEM((_BPW,), jnp.int32),
        pltpu.VMEM((_C, _DIM), jnp.float32),
        pltpu.VMEM((_C, _DIM), jnp.float32),
        pltpu.VMEM((_C * _DIM,), jnp.float32),
        pltpu.SemaphoreType.DMA,
        pltpu.SemaphoreType.DMA,
        pltpu.SemaphoreType.DMA,
    ],
    compiler_params=pltpu.CompilerParams(
        use_tc_tiling_on_sc=False, needs_layout_passes=False),
)
def _gather_kernel(idx_hbm, table_hbm, out_hbm, idx_v, buf0, buf1, abuf,
                   sg0, sg1, ss):
    wid = lax.axis_index("s") * _NC + lax.axis_index("c")
    base = wid * _BPW
    pltpu.sync_copy(idx_hbm.at[pl.ds(base, _BPW)], idx_v)
    bufs = (buf0, buf1)
    gsems = (sg0, sg1)
    lane = lax.broadcasted_iota(jnp.int32, (16,), 0)

    def transpose_chunk(gbuf):
        @plsc.parallel_loop(0, _C // 8, unroll=1)
        def _(u):
            t = u >> 1
            j0 = (u & 1) * (_DIM // 2)
            rows = t * 16 + lane
            abase = t * 16 + (t >> 3) * (_BLK * _DIM - _BLK)
            for jj in range(_DIM // 2):
                colv = (j0 + jj + lane) & (_DIM - 1)
                v = plsc.load_gather(gbuf, [rows, colv])
                plsc.store_scatter(abuf, [abase + (colv << 7) + lane], v)

    def store_chunk(g):
        beta0 = wid * _BPWB + g * _MPC
        copies = []
        for m in range(_MPC):
            beta = beta0 + m
            f = beta // _NBB
            c = beta - f * _NBB
            for k in range(_KT):
                off = ((f * _KT + k) * _NBB + c) * (8 * _BLK)
                copies.append(pltpu.async_copy(
                    abuf.at[pl.ds(m * (_BLK * _DIM) + k * (8 * _BLK),
                                  8 * _BLK)],
                    out_hbm.at[pl.ds(off, 8 * _BLK)], ss))
        return copies

    gathers = [None] * _NCHUNK
    stores = [None] * _NCHUNK
    gathers[0] = pltpu.async_copy(
        table_hbm.at[idx_v.at[pl.ds(0, _C)]], bufs[0], sg0)
    for g in range(_NCHUNK):
        nxt = g + 1
        if nxt < _NCHUNK:
            gathers[nxt] = pltpu.async_copy(
                table_hbm.at[idx_v.at[pl.ds(nxt * _C, _C)]],
                bufs[nxt % 2], gsems[nxt % 2])
        gathers[g].wait()
        if g >= 1:
            for cp in stores[g - 1]:
                cp.wait()
        transpose_chunk(bufs[g % 2])
        stores[g] = store_chunk(g)
    for cp in stores[_NCHUNK - 1]:
        cp.wait()


def kernel(x, weights):
    idx = x.T.reshape(_B)
    tail = weights[_NSLAB * _SLABC:].reshape(-1)
    tlin = _relayout_kernel(weights.T, tail).reshape(_NEMB, _DIM)
    flat = _gather_kernel(idx, tlin)
    out5 = flat.reshape(_N_FIELDS, _KT, _NBB, 8, _BLK)
    return out5.transpose(2, 4, 0, 1, 3).reshape(_BATCH, _N_FIELDS, _DIM)
